# Initial kernel scaffold; baseline (speedup 1.0000x reference)
import jax
import jax.numpy as jnp
from jax.experimental import pallas as pl


def kernel(x_feat, segment_ids, num_segments):
    raise NotImplementedError("write your pallas kernel here")


# speedup vs baseline: 7.9694x; 7.9694x over previous
import functools

import jax
import jax.numpy as jnp
from jax import lax
from jax.experimental import pallas as pl
from jax.experimental.pallas import tpu as pltpu
from jax.experimental.pallas import tpu_sc as plsc

N = 320000
D = 128
SEGS = 16
EPS = 1e-06

NC = 2
NS = 16
NW = NC * NS
ROWS_PER_TILE = N // NW
R = 400
NCHUNK = ROWS_PER_TILE // R
GROUPS = R // 16
DCH = D // 16

_mesh = plsc.VectorSubcoreMesh(core_axis_name="c", subcore_axis_name="s")


@functools.partial(
    pl.kernel,
    out_type=[
        jax.ShapeDtypeStruct((NW, SEGS, D), jnp.float32),
        jax.ShapeDtypeStruct((NW, SEGS), jnp.float32),
    ],
    mesh=_mesh,
    scratch_types=[
        pltpu.VMEM((R, D), jnp.float32),
        pltpu.VMEM((R,), jnp.int32),
        pltpu.VMEM((SEGS, D), jnp.float32),
        pltpu.VMEM((SEGS,), jnp.float32),
    ],
)
def _seg_sums(x_hbm, ids_hbm, psum_hbm, pcnt_hbm, xbuf, idbuf, acc, cntv):
    cid = lax.axis_index("c")
    sid = lax.axis_index("s")
    wid = sid * NC + cid
    base = wid * ROWS_PER_TILE

    zero16 = jnp.zeros((16,), jnp.float32)
    lanes = lax.iota(jnp.int32, 16)

    def zero_acc(s, carry):
        for c in range(DCH):
            acc[s, pl.ds(c * 16, 16)] = zero16
        return carry

    lax.fori_loop(0, SEGS, zero_acc, 0)
    cntv[...] = zero16

    def accum_row(r, seg):
        for c in range(DCH):
            v = jnp.maximum(xbuf[r, pl.ds(c * 16, 16)], EPS)
            acc[seg, pl.ds(c * 16, 16)] += v

    def process_group(g, carry):
        r0 = g * 16
        ids = idbuf[pl.ds(r0, 16)]
        ids_s = [ids[j] for j in range(16)]
        mn = functools.reduce(jnp.minimum, ids_s)
        mx = functools.reduce(jnp.maximum, ids_s)

        @pl.when(mn == mx)
        def _common():
            sums = [zero16 for _ in range(DCH)]
            for j in range(16):
                for c in range(DCH):
                    v = jnp.maximum(xbuf[r0 + j, pl.ds(c * 16, 16)], EPS)
                    sums[c] = sums[c] + v
            for c in range(DCH):
                acc[mn, pl.ds(c * 16, 16)] += sums[c]
            cntv[...] += jnp.where(lanes == mn, 16.0, 0.0)

        @pl.when(mn != mx)
        def _boundary():
            for j in range(16):
                sj = ids_s[j]
                accum_row(r0 + j, sj)
                cntv[...] += jnp.where(lanes == sj, 1.0, 0.0)

        return carry

    def chunk_body(i, carry):
        row0 = base + i * R
        pltpu.sync_copy(x_hbm.at[pl.ds(row0, R)], xbuf)
        pltpu.sync_copy(ids_hbm.at[pl.ds(row0, R)], idbuf)
        lax.fori_loop(0, GROUPS, process_group, 0)
        return carry

    lax.fori_loop(0, NCHUNK, chunk_body, 0)

    pltpu.sync_copy(acc, psum_hbm.at[wid])
    pltpu.sync_copy(cntv, pcnt_hbm.at[wid])


def _combine_body(psum_ref, pcnt_ref, out_ref):
    sums = jnp.sum(psum_ref[...], axis=0)
    cnts = jnp.sum(pcnt_ref[...], axis=0)
    out_ref[...] = sums / jnp.maximum(cnts, 1.0)[:, None]


_combine = pl.pallas_call(
    _combine_body,
    out_shape=jax.ShapeDtypeStruct((SEGS, D), jnp.float32),
)


def kernel(x_feat, segment_ids, num_segments):
    psum, pcnt = _seg_sums(x_feat, segment_ids.astype(jnp.int32))
    return _combine(psum, pcnt)


# speedup vs baseline: 13.0479x; 1.6373x over previous
import functools

import jax
import jax.numpy as jnp
from jax import lax
from jax.experimental import pallas as pl
from jax.experimental.pallas import tpu as pltpu
from jax.experimental.pallas import tpu_sc as plsc

N = 320000
D = 128
SEGS = 16
EPS = 1e-06

NC = 2
NS = 16
NW = NC * NS
ROWS_PER_TILE = N // NW
R = 400
NCHUNK = ROWS_PER_TILE // R
GROUPS = R // 16
DCH = D // 16

_mesh = plsc.VectorSubcoreMesh(core_axis_name="c", subcore_axis_name="s")


@functools.partial(
    pl.kernel,
    out_type=[
        jax.ShapeDtypeStruct((NW, SEGS, D), jnp.float32),
        jax.ShapeDtypeStruct((NW, SEGS), jnp.float32),
    ],
    mesh=_mesh,
    scratch_types=[
        pltpu.VMEM((R, D), jnp.float32),
        pltpu.VMEM((R, D), jnp.float32),
        pltpu.VMEM((R,), jnp.int32),
        pltpu.VMEM((R,), jnp.int32),
        pltpu.VMEM((SEGS, D), jnp.float32),
        pltpu.VMEM((SEGS,), jnp.float32),
        pltpu.SemaphoreType.DMA,
        pltpu.SemaphoreType.DMA,
    ],
)
def _seg_sums(x_hbm, ids_hbm, psum_hbm, pcnt_hbm,
              xbuf0, xbuf1, idbuf0, idbuf1, acc, cntv, sem0, sem1):
    cid = lax.axis_index("c")
    sid = lax.axis_index("s")
    wid = sid * NC + cid
    base = wid * ROWS_PER_TILE

    zero16 = jnp.zeros((16,), jnp.float32)
    lanes = lax.iota(jnp.int32, 16)

    def zero_acc(s, carry):
        for c in range(DCH):
            acc[s, pl.ds(c * 16, 16)] = zero16
        return carry

    lax.fori_loop(0, SEGS, zero_acc, 0)
    cntv[...] = zero16

    def make_group_processor(xbuf, idbuf):
        def accum_row(r, seg):
            for c in range(DCH):
                v = jnp.maximum(xbuf[r, pl.ds(c * 16, 16)], EPS)
                acc[seg, pl.ds(c * 16, 16)] += v

        def process_group(g, carry):
            r0 = g * 16
            ids = idbuf[pl.ds(r0, 16)]
            ids_s = [ids[j] for j in range(16)]
            mn = functools.reduce(jnp.minimum, ids_s)
            mx = functools.reduce(jnp.maximum, ids_s)

            @pl.when(mn == mx)
            def _common():
                sums = [zero16 for _ in range(DCH)]
                for j in range(16):
                    for c in range(DCH):
                        v = jnp.maximum(xbuf[r0 + j, pl.ds(c * 16, 16)], EPS)
                        sums[c] = sums[c] + v
                for c in range(DCH):
                    acc[mn, pl.ds(c * 16, 16)] += sums[c]
                cntv[...] += jnp.where(lanes == mn, 16.0, 0.0)

            @pl.when(mn != mx)
            def _boundary():
                for j in range(16):
                    sj = ids_s[j]
                    accum_row(r0 + j, sj)
                    cntv[...] += jnp.where(lanes == sj, 1.0, 0.0)

            return carry

        return process_group

    process0 = make_group_processor(xbuf0, idbuf0)
    process1 = make_group_processor(xbuf1, idbuf1)

    def start(i, xbuf, idbuf, sem):
        row0 = base + i * R
        pltpu.async_copy(x_hbm.at[pl.ds(row0, R)], xbuf, sem)
        pltpu.async_copy(ids_hbm.at[pl.ds(row0, R)], idbuf, sem)

    def wait(i, xbuf, idbuf, sem):
        row0 = base + i * R
        pltpu.make_async_copy(x_hbm.at[pl.ds(row0, R)], xbuf, sem).wait()
        pltpu.make_async_copy(ids_hbm.at[pl.ds(row0, R)], idbuf, sem).wait()

    start(0, xbuf0, idbuf0, sem0)

    def pair_body(p, carry):
        i0 = 2 * p
        start(i0 + 1, xbuf1, idbuf1, sem1)
        wait(i0, xbuf0, idbuf0, sem0)
        lax.fori_loop(0, GROUPS, process0, 0)
        start(i0 + 2, xbuf0, idbuf0, sem0)
        wait(i0 + 1, xbuf1, idbuf1, sem1)
        lax.fori_loop(0, GROUPS, process1, 0)
        return carry

    lax.fori_loop(0, (NCHUNK - 1) // 2, pair_body, 0)
    wait(NCHUNK - 1, xbuf0, idbuf0, sem0)
    lax.fori_loop(0, GROUPS, process0, 0)

    pltpu.sync_copy(acc, psum_hbm.at[wid])
    pltpu.sync_copy(cntv, pcnt_hbm.at[wid])


def _combine_body(psum_ref, pcnt_ref, out_ref):
    sums = jnp.sum(psum_ref[...], axis=0)
    cnts = jnp.sum(pcnt_ref[...], axis=0)
    out_ref[...] = sums / jnp.maximum(cnts, 1.0)[:, None]


_combine = pl.pallas_call(
    _combine_body,
    out_shape=jax.ShapeDtypeStruct((SEGS, D), jnp.float32),
)


def kernel(x_feat, segment_ids, num_segments):
    psum, pcnt = _seg_sums(x_feat, segment_ids.astype(jnp.int32))
    return _combine(psum, pcnt)


# speedup vs baseline: 14.5405x; 1.1144x over previous
import functools

import jax
import jax.numpy as jnp
from jax import lax
from jax.experimental import pallas as pl
from jax.experimental.pallas import tpu as pltpu
from jax.experimental.pallas import tpu_sc as plsc

N = 320000
D = 128
SEGS = 16
EPS = 1e-06

NC = 2
NS = 16
NW = NC * NS
ROWS_PER_TILE = N // NW
R = 400
NCHUNK = ROWS_PER_TILE // R
GROUPS = R // 16
DCH = D // 16

_mesh = plsc.VectorSubcoreMesh(core_axis_name="c", subcore_axis_name="s")


@functools.partial(
    pl.kernel,
    out_type=[
        jax.ShapeDtypeStruct((NW, SEGS, D), jnp.float32),
        jax.ShapeDtypeStruct((NW, SEGS), jnp.float32),
    ],
    mesh=_mesh,
    scratch_types=[
        pltpu.VMEM((R, D), jnp.float32),
        pltpu.VMEM((R, D), jnp.float32),
        pltpu.VMEM((R,), jnp.int32),
        pltpu.VMEM((R,), jnp.int32),
        pltpu.VMEM((SEGS, D), jnp.float32),
        pltpu.VMEM((SEGS,), jnp.float32),
        pltpu.SemaphoreType.DMA,
        pltpu.SemaphoreType.DMA,
    ],
)
def _seg_sums(x_hbm, ids_hbm, psum_hbm, pcnt_hbm,
              xbuf0, xbuf1, idbuf0, idbuf1, acc, cntv, sem0, sem1):
    cid = lax.axis_index("c")
    sid = lax.axis_index("s")
    wid = sid * NC + cid
    base = wid * ROWS_PER_TILE

    zero16 = jnp.zeros((16,), jnp.float32)
    lanes = lax.iota(jnp.int32, 16)

    def zero_acc(s, carry):
        for c in range(DCH):
            acc[s, pl.ds(c * 16, 16)] = zero16
        return carry

    lax.fori_loop(0, SEGS, zero_acc, 0)
    cntv[...] = zero16

    def make_group_processor(xbuf, idbuf):
        def accum_row(r, seg):
            for c in range(DCH):
                v = jnp.maximum(xbuf[r, pl.ds(c * 16, 16)], EPS)
                acc[seg, pl.ds(c * 16, 16)] += v

        def process_group(g, carry):
            r0 = g * 16
            ids = idbuf[pl.ds(r0, 16)]
            mn = ids[0]
            mx = ids[15]

            @pl.when(mn == mx)
            def _common():
                sums = [zero16 for _ in range(DCH)]
                for j in range(16):
                    for c in range(DCH):
                        v = jnp.maximum(xbuf[r0 + j, pl.ds(c * 16, 16)], EPS)
                        sums[c] = sums[c] + v
                for c in range(DCH):
                    acc[mn, pl.ds(c * 16, 16)] += sums[c]
                cntv[...] += jnp.where(lanes == mn, 16.0, 0.0)

            @pl.when(mn != mx)
            def _boundary():
                for j in range(16):
                    sj = ids[j]
                    accum_row(r0 + j, sj)
                    cntv[...] += jnp.where(lanes == sj, 1.0, 0.0)

            return carry

        return process_group

    process0 = make_group_processor(xbuf0, idbuf0)
    process1 = make_group_processor(xbuf1, idbuf1)

    def start(i, xbuf, idbuf, sem):
        row0 = base + i * R
        pltpu.async_copy(x_hbm.at[pl.ds(row0, R)], xbuf, sem)
        pltpu.async_copy(ids_hbm.at[pl.ds(row0, R)], idbuf, sem)

    def wait(i, xbuf, idbuf, sem):
        row0 = base + i * R
        pltpu.make_async_copy(x_hbm.at[pl.ds(row0, R)], xbuf, sem).wait()
        pltpu.make_async_copy(ids_hbm.at[pl.ds(row0, R)], idbuf, sem).wait()

    start(0, xbuf0, idbuf0, sem0)

    def pair_body(p, carry):
        i0 = 2 * p
        start(i0 + 1, xbuf1, idbuf1, sem1)
        wait(i0, xbuf0, idbuf0, sem0)
        lax.fori_loop(0, GROUPS, process0, 0)
        start(i0 + 2, xbuf0, idbuf0, sem0)
        wait(i0 + 1, xbuf1, idbuf1, sem1)
        lax.fori_loop(0, GROUPS, process1, 0)
        return carry

    lax.fori_loop(0, (NCHUNK - 1) // 2, pair_body, 0)
    wait(NCHUNK - 1, xbuf0, idbuf0, sem0)
    lax.fori_loop(0, GROUPS, process0, 0)

    pltpu.sync_copy(acc, psum_hbm.at[wid])
    pltpu.sync_copy(cntv, pcnt_hbm.at[wid])


def _combine_body(psum_ref, pcnt_ref, out_ref):
    sums = jnp.sum(psum_ref[...], axis=0)
    cnts = jnp.sum(pcnt_ref[...], axis=0)
    out_ref[...] = sums / jnp.maximum(cnts, 1.0)[:, None]


_combine = pl.pallas_call(
    _combine_body,
    out_shape=jax.ShapeDtypeStruct((SEGS, D), jnp.float32),
)


def kernel(x_feat, segment_ids, num_segments):
    psum, pcnt = _seg_sums(x_feat, segment_ids.astype(jnp.int32))
    return _combine(psum, pcnt)
